# Initial kernel scaffold; baseline (speedup 1.0000x reference)
import jax
import jax.numpy as jnp
from jax.experimental import pallas as pl


def kernel(hidden_states, gate_w, e_bias, w_gate_up, w_down, s_gate_up, s_down):
    raise NotImplementedError("write your pallas kernel here")


# speedup vs baseline: 1.1357x; 1.1357x over previous
import functools

import jax
import jax.numpy as jnp
from jax.experimental import pallas as pl
from jax.experimental.pallas import tpu as pltpu

T = 2048
D = 2048
E = 8
TOPK = 2
I_MOE = 1024
I_SHARED = 2048

TB = 256
RB = 256
NTB = T // TB
NA = T * TOPK
R_PAD = 5888
NB = R_PAD // RB


def _stage_a_body(x_ref, gate_ref, ebias_ref, sgu_ref, sdn_ref,
                  shared_ref, route_ref):
    x = x_ref[...]
    logits = jax.lax.dot_general(
        x, gate_ref[...], (((1,), (1,)), ((), ())),
        preferred_element_type=jnp.float32)
    scores = jax.nn.sigmoid(logits)
    sfc = scores + ebias_ref[...]
    lane = jax.lax.broadcasted_iota(jnp.int32, (TB, E), 1)
    big = jnp.float32(1e30)
    m1 = jnp.max(sfc, axis=1, keepdims=True)
    i1 = jnp.min(jnp.where(sfc == m1, lane, E), axis=1, keepdims=True)
    oh1 = lane == i1
    sfc2 = jnp.where(oh1, -big, sfc)
    m2 = jnp.max(sfc2, axis=1, keepdims=True)
    i2 = jnp.min(jnp.where(sfc2 == m2, lane, E), axis=1, keepdims=True)
    oh2 = lane == i2
    w1 = jnp.sum(jnp.where(oh1, scores, 0.0), axis=1, keepdims=True)
    w2 = jnp.sum(jnp.where(oh2, scores, 0.0), axis=1, keepdims=True)
    denom = w1 + w2
    w1n = w1 / denom
    w2n = w2 / denom
    route = jnp.where(
        lane == 0, i1.astype(jnp.float32),
        jnp.where(lane == 1, i2.astype(jnp.float32),
                  jnp.where(lane == 2, w1n,
                            jnp.where(lane == 3, w2n, 0.0))))
    route_ref[...] = route

    xb = x.astype(jnp.bfloat16)
    gu = jax.lax.dot_general(
        xb, sgu_ref[...], (((1,), (1,)), ((), ())),
        preferred_element_type=jnp.float32)
    a = gu[:, :I_SHARED]
    b = gu[:, I_SHARED:]
    h = (a * jax.nn.sigmoid(a) * b).astype(jnp.bfloat16)
    shared_ref[...] = jax.lax.dot_general(
        h, sdn_ref[...], (((1,), (1,)), ((), ())),
        preferred_element_type=jnp.float32)


def _stage_b_body(be_ref, x_ref, wgu_ref, wdn_ref, y_ref):
    xb = x_ref[...]
    gu = jax.lax.dot_general(
        xb, wgu_ref[0], (((1,), (1,)), ((), ())),
        preferred_element_type=jnp.float32)
    a = gu[:, :I_MOE]
    b = gu[:, I_MOE:]
    h = (a * jax.nn.sigmoid(a) * b).astype(jnp.bfloat16)
    y_ref[...] = jax.lax.dot_general(
        h, wdn_ref[0], (((1,), (1,)), ((), ())),
        preferred_element_type=jnp.float32)


def kernel(hidden_states, gate_w, e_bias, w_gate_up, w_down, s_gate_up, s_down):
    x = hidden_states
    sgu = s_gate_up.astype(jnp.bfloat16)
    sdn = s_down.astype(jnp.bfloat16)
    wgu = w_gate_up.astype(jnp.bfloat16)
    wdn = w_down.astype(jnp.bfloat16)

    shared, route = pl.pallas_call(
        _stage_a_body,
        grid=(NTB,),
        in_specs=[
            pl.BlockSpec((TB, D), lambda i: (i, 0)),
            pl.BlockSpec((E, D), lambda i: (0, 0)),
            pl.BlockSpec((1, E), lambda i: (0, 0)),
            pl.BlockSpec((2 * I_SHARED, D), lambda i: (0, 0)),
            pl.BlockSpec((D, I_SHARED), lambda i: (0, 0)),
        ],
        out_specs=[
            pl.BlockSpec((TB, D), lambda i: (i, 0)),
            pl.BlockSpec((TB, E), lambda i: (i, 0)),
        ],
        out_shape=[
            jax.ShapeDtypeStruct((T, D), jnp.float32),
            jax.ShapeDtypeStruct((T, E), jnp.float32),
        ],
        compiler_params=pltpu.CompilerParams(
            dimension_semantics=("arbitrary",)),
    )(x, gate_w, e_bias.reshape(1, E), sgu, sdn)

    topk_idx = route[:, :TOPK].astype(jnp.int32)
    topk_w = route[:, TOPK:2 * TOPK]

    ids = topk_idx.reshape(-1)
    order = jnp.argsort(ids, stable=True).astype(jnp.int32)
    ids_sorted = ids[order]
    counts = jnp.zeros((E,), jnp.int32).at[ids].add(1)
    padded = ((counts + RB - 1) // RB) * RB
    pstart = jnp.concatenate([jnp.zeros((1,), jnp.int32),
                              jnp.cumsum(padded)])[:E]
    cstart = jnp.concatenate([jnp.zeros((1,), jnp.int32),
                              jnp.cumsum(counts)])[:E]
    rank = jnp.arange(NA, dtype=jnp.int32) - cstart[ids_sorted]
    dest_sorted = pstart[ids_sorted] + rank
    srctid = jnp.zeros((R_PAD,), jnp.int32).at[dest_sorted].set(order // TOPK)
    dpos = jnp.zeros((NA,), jnp.int32).at[order].set(
        dest_sorted.astype(jnp.int32))
    dpos = dpos.reshape(T, TOPK)
    ends = jnp.cumsum(padded)
    block_expert = jnp.minimum(
        jnp.searchsorted(ends, jnp.arange(NB, dtype=jnp.int32) * RB,
                         side="right").astype(jnp.int32), E - 1)

    xb = x.astype(jnp.bfloat16)
    x_sorted = jnp.take(xb, srctid, axis=0)

    y = pl.pallas_call(
        _stage_b_body,
        grid_spec=pltpu.PrefetchScalarGridSpec(
            num_scalar_prefetch=1,
            grid=(NB,),
            in_specs=[
                pl.BlockSpec((RB, D), lambda b, be: (b, 0)),
                pl.BlockSpec((1, 2 * I_MOE, D), lambda b, be: (be[b], 0, 0)),
                pl.BlockSpec((1, D, I_MOE), lambda b, be: (be[b], 0, 0)),
            ],
            out_specs=pl.BlockSpec((RB, D), lambda b, be: (b, 0)),
        ),
        out_shape=jax.ShapeDtypeStruct((R_PAD, D), jnp.float32),
        compiler_params=pltpu.CompilerParams(
            dimension_semantics=("arbitrary",)),
    )(block_expert, x_sorted, wgu, wdn)

    y0 = jnp.take(y, dpos[:, 0], axis=0)
    y1 = jnp.take(y, dpos[:, 1], axis=0)
    return shared + topk_w[:, :1] * y0 + topk_w[:, 1:] * y1
